# baseline (speedup 1.0000x reference)
import functools

import jax
import jax.numpy as jnp
from jax import lax
from jax.experimental import pallas as pl
from jax.experimental.pallas import tpu as pltpu
from jax.experimental.pallas import tpu_sc as plsc

B = 16384
D = 32
NC = 2
NS = 16
NW = NC * NS
BPW = B // NW
CHUNK = 128
NCH = BPW // CHUNK

_sc_mesh = plsc.VectorSubcoreMesh(core_axis_name="c", subcore_axis_name="s")


@functools.partial(
    pl.kernel,
    mesh=_sc_mesh,
    compiler_params=pltpu.CompilerParams(use_tc_tiling_on_sc=False),
    out_type=(
        jax.ShapeDtypeStruct((B, D), jnp.float32),
        jax.ShapeDtypeStruct((B, D), jnp.float32),
        jax.ShapeDtypeStruct((B, D), jnp.float32),
        jax.ShapeDtypeStruct((B, D), jnp.float32),
    ),
    scratch_types=(
        pltpu.VMEM((NCH, CHUNK), jnp.int32),
        pltpu.VMEM((NCH, CHUNK), jnp.int32),
        pltpu.VMEM((BPW, D), jnp.float32),
        pltpu.VMEM((BPW, D), jnp.float32),
        pltpu.VMEM((BPW, D), jnp.float32),
        pltpu.VMEM((BPW, D), jnp.float32),
        pltpu.SemaphoreType.DMA,
    ),
)
def _gather_sc(uidx_hbm, iidx_hbm, mfu_hbm, mfi_hbm, mlu_hbm, mli_hbm,
               out_mfu, out_mfi, out_mlu, out_mli,
               uidx_v, iidx_v, mfu_v, mfi_v, mlu_v, mli_v, sem):
    wid = lax.axis_index("s") * NC + lax.axis_index("c")
    row0 = wid * NCH
    base = wid * BPW
    pltpu.sync_copy(uidx_hbm.at[pl.ds(row0, NCH)], uidx_v)
    pltpu.sync_copy(iidx_hbm.at[pl.ds(row0, NCH)], iidx_v)
    descs = []
    for j in range(NCH):
        sl = pl.ds(j * CHUNK, CHUNK)
        descs.append(pltpu.async_copy(mfu_hbm.at[uidx_v.at[j]], mfu_v.at[sl], sem))
        descs.append(pltpu.async_copy(mfi_hbm.at[iidx_v.at[j]], mfi_v.at[sl], sem))
        descs.append(pltpu.async_copy(mlu_hbm.at[uidx_v.at[j]], mlu_v.at[sl], sem))
        descs.append(pltpu.async_copy(mli_hbm.at[iidx_v.at[j]], mli_v.at[sl], sem))
    for d in descs:
        d.wait()
    pltpu.sync_copy(mfu_v, out_mfu.at[pl.ds(base, BPW)])
    pltpu.sync_copy(mfi_v, out_mfi.at[pl.ds(base, BPW)])
    pltpu.sync_copy(mlu_v, out_mlu.at[pl.ds(base, BPW)])
    pltpu.sync_copy(mli_v, out_mli.at[pl.ds(base, BPW)])


BB = 2048


def _dense_tc(mfu_ref, mfi_ref, mlu_ref, mli_ref,
              w1u_ref, w1i_ref, b1_ref, w2t_ref, b2_ref,
              wpm_ref, wph_ref, bp_ref, out_ref):
    h1 = jnp.dot(mlu_ref[...], w1u_ref[...], preferred_element_type=jnp.float32)
    h1 = h1 + jnp.dot(mli_ref[...], w1i_ref[...], preferred_element_type=jnp.float32)
    h1 = jnp.maximum(h1 + b1_ref[...], 0.0)
    h2 = jnp.dot(h1, w2t_ref[...], preferred_element_type=jnp.float32)
    h2 = jnp.maximum(h2 + b2_ref[...], 0.0)
    mf = mfu_ref[...] * mfi_ref[...]
    acc = jnp.sum(mf * wpm_ref[...], axis=1) + jnp.sum(h2 * wph_ref[...], axis=1)
    out_ref[...] = acc + bp_ref[0, 0]


def kernel(user_idx, item_idx, mf_user_w, mf_item_w, mlp_user_w, mlp_item_w,
           W1, b1, W2, b2, Wp, bp):
    uidx = user_idx.astype(jnp.int32).reshape(B // CHUNK, CHUNK)
    iidx = item_idx.astype(jnp.int32).reshape(B // CHUNK, CHUNK)

    mfu, mfi, mlu, mli = _gather_sc(uidx, iidx, mf_user_w, mf_item_w,
                                    mlp_user_w, mlp_item_w)

    w1u = W1[:, :D].T
    w1i = W1[:, D:].T
    w2t = W2.T
    b1r = b1.reshape(1, -1)
    b2r = b2.reshape(1, -1)
    wpm = Wp[:, :D]
    wph = Wp[:, D:]
    bpr = bp.reshape(1, 1)

    grid = B // BB
    full = lambda i: (0, 0)
    out = pl.pallas_call(
        _dense_tc,
        grid=(grid,),
        in_specs=[
            pl.BlockSpec((BB, D), lambda i: (i, 0)),
            pl.BlockSpec((BB, D), lambda i: (i, 0)),
            pl.BlockSpec((BB, D), lambda i: (i, 0)),
            pl.BlockSpec((BB, D), lambda i: (i, 0)),
            pl.BlockSpec((D, 32), full),
            pl.BlockSpec((D, 32), full),
            pl.BlockSpec((1, 32), full),
            pl.BlockSpec((D, 16), full),
            pl.BlockSpec((1, 16), full),
            pl.BlockSpec((1, D), full),
            pl.BlockSpec((1, 16), full),
            pl.BlockSpec((1, 1), full),
        ],
        out_specs=pl.BlockSpec((BB,), lambda i: (i,)),
        out_shape=jax.ShapeDtypeStruct((B,), jnp.float32),
    )(mfu, mfi, mlu, mli, w1u, w1i, b1r, w2t, b2r, wpm, wph, bpr)
    return out
